# baseline (speedup 1.0000x reference)
import functools

import jax
import jax.numpy as jnp
from jax import lax
from jax.experimental import pallas as pl
from jax.experimental.pallas import tpu as pltpu
from jax.experimental.pallas import tpu_sc as plsc

NUM_GRAPHS = 64
REUSE_WEIGHT = 0.01

NC = 2
NS = 16
L = 16
NW = NC * NS
ROWS = 100000
D = 128
RPW = ROWS // NW
CHUNK = 125
NCHUNK = RPW // CHUNK
NCHUNK_PAD = 2 * L
SEG_PAD = NUM_GRAPHS + 1
NBUF = 6
RUNROLL = 5


def _seg_body(codes_hbm, batch_hbm, sums_out, cnts_out,
              idx_v, bufs, ones_v, csum_v, c125_v, fid_v,
              sums_sh, cnts_sh, load_sems):
    c = lax.axis_index("c")
    s = lax.axis_index("s")
    wid = s * NC + c
    base = wid * RPW

    zvec = jnp.zeros((L,), jnp.float32)

    @pl.when(s == 0)
    def _init():
        def zrow(i, carry):
            for jj in range(D // L):
                bufs[0, i, pl.ds(jj * L, L)] = zvec
            ones_v[i, :] = zvec
            return carry
        lax.fori_loop(0, SEG_PAD, zrow, 0)
        pltpu.sync_copy(bufs.at[0].at[pl.ds(0, SEG_PAD)], sums_sh)
        pltpu.sync_copy(ones_v.at[pl.ds(0, SEG_PAD)], cnts_sh)

    plsc.subcore_barrier()

    ovec = jnp.ones((L,), jnp.float32)

    def orow(i, carry):
        ones_v[i, :] = ovec
        return carry
    lax.fori_loop(0, CHUNK, orow, 0)

    cvec = jnp.full((L,), float(CHUNK), jnp.float32)
    for i in range(NCHUNK_PAD):
        c125_v[i, :] = cvec

    def load(j):
        pltpu.async_copy(codes_hbm.at[pl.ds(base + j * CHUNK, CHUNK)],
                         bufs.at[j % NBUF],
                         load_sems.at[j % NBUF])

    def wait_load(j):
        pltpu.make_async_copy(codes_hbm.at[pl.ds(base + j * CHUNK, CHUNK)],
                              bufs.at[j % NBUF],
                              load_sems.at[j % NBUF]).wait()

    for j in range(NBUF - 1):
        load(j)
    pltpu.sync_copy(batch_hbm.at[pl.ds(wid * NCHUNK, NCHUNK)], idx_v)

    lane_iota = lax.iota(jnp.int32, L)
    mark = jnp.full((L,), NUM_GRAPHS, jnp.int32)
    fids = [None, mark]

    for j in range(NCHUNK):
        if j + NBUF - 1 < NCHUNK:
            load(j + NBUF - 1)
        wait_load(j)
        buf = bufs.at[j % NBUF]

        vmin = idx_v[j, pl.ds(0, L)]
        vmax = vmin
        for off in (16, 32, 48, 64, 80, 96, 109):
            sl = idx_v[j, pl.ds(off, L)]
            vmin = jnp.minimum(vmin, sl)
            vmax = jnp.maximum(vmax, sl)
        smin = lax.reduce_min(vmin, axes=(0,))
        smax = lax.reduce_max(vmax, axes=(0,))
        mixed = smin != smax
        fid_j = jnp.where(mixed, NUM_GRAPHS, smin)
        half, lane = divmod(j, L)
        if fids[half] is None:
            fids[half] = jnp.where(lane_iota == lane, fid_j, 0)
        else:
            fids[half] = jnp.where(lane_iota == lane, fid_j, fids[half])

        @pl.when(mixed)
        def _fallback():
            pltpu.sync_copy(buf, sums_sh.at[idx_v.at[j]], add=True)
            pltpu.sync_copy(ones_v, cnts_sh.at[idx_v.at[j]], add=True)

        @pl.when(jnp.logical_not(mixed))
        def _reduce():
            def rbody(r5, accs):
                accs = list(accs)
                for rr in range(RUNROLL):
                    r = r5 * RUNROLL + rr
                    for jj in range(D // L):
                        accs[jj] = accs[jj] + buf[r, pl.ds(jj * L, L)]
                return tuple(accs)
            accs = lax.fori_loop(0, CHUNK // RUNROLL, rbody,
                                 tuple(zvec for _ in range(D // L)))
            for jj in range(D // L):
                csum_v[j, pl.ds(jj * L, L)] = accs[jj]

    fid_v[pl.ds(0, L)] = fids[0]
    fid_v[pl.ds(L, L)] = fids[1]
    pltpu.sync_copy(csum_v, sums_sh.at[fid_v], add=True)
    pltpu.sync_copy(c125_v, cnts_sh.at[fid_v], add=True)

    plsc.subcore_barrier()

    @pl.when(s == 0)
    def _flush():
        pltpu.sync_copy(sums_sh, bufs.at[0].at[pl.ds(0, SEG_PAD)])
        pltpu.sync_copy(bufs.at[0].at[pl.ds(0, SEG_PAD)], sums_out.at[c])
        pltpu.sync_copy(cnts_sh, ones_v.at[pl.ds(0, SEG_PAD)])
        pltpu.sync_copy(ones_v.at[pl.ds(0, SEG_PAD)], cnts_out.at[c])


@functools.lru_cache(maxsize=1)
def _make_seg_reduce():
    return functools.partial(
        pl.kernel,
        out_type=[
            jax.ShapeDtypeStruct((NC, SEG_PAD, D), jnp.float32),
            jax.ShapeDtypeStruct((NC, SEG_PAD, L), jnp.float32),
        ],
        mesh=plsc.VectorSubcoreMesh(core_axis_name="c", subcore_axis_name="s"),
        scratch_types=[
            pltpu.VMEM((NCHUNK, CHUNK), jnp.int32),
            pltpu.VMEM((NBUF, CHUNK, D), jnp.float32),
            pltpu.VMEM((CHUNK, L), jnp.float32),
            pltpu.VMEM((NCHUNK_PAD, D), jnp.float32),
            pltpu.VMEM((NCHUNK_PAD, L), jnp.float32),
            pltpu.VMEM((NCHUNK_PAD,), jnp.int32),
            pltpu.VMEM_SHARED((SEG_PAD, D), jnp.float32),
            pltpu.VMEM_SHARED((SEG_PAD, L), jnp.float32),
            pltpu.SemaphoreType.DMA((NBUF,)),
        ],
        compiler_params=pltpu.CompilerParams(use_tc_tiling_on_sc=False,
                                             needs_layout_passes=False),
    )(_seg_body)


def _fin_body(s_ref, c_ref, o_ref):
    sums = s_ref[0, :NUM_GRAPHS, :] + s_ref[1, :NUM_GRAPHS, :]
    counts = c_ref[0, :NUM_GRAPHS, 0:1] + c_ref[1, :NUM_GRAPHS, 0:1]
    means = sums / counts
    mu = jnp.mean(means, axis=0, keepdims=True)
    dev = means - mu
    var = jnp.sum(dev * dev, axis=0) / (NUM_GRAPHS - 1)
    o_ref[...] = jnp.reshape(-REUSE_WEIGHT * jnp.mean(var), (1, 1))


def kernel(sparse_codes, batch):
    batch2d = batch.astype(jnp.int32).reshape(NW * NCHUNK, CHUNK)
    sums, cnts = _make_seg_reduce()(sparse_codes, batch2d)
    out = pl.pallas_call(
        _fin_body,
        out_shape=jax.ShapeDtypeStruct((1, 1), jnp.float32),
    )(sums, cnts)
    return out[0, 0]


# speedup vs baseline: 1.0196x; 1.0196x over previous
import functools

import jax
import jax.numpy as jnp
from jax import lax
from jax.experimental import pallas as pl
from jax.experimental.pallas import tpu as pltpu
from jax.experimental.pallas import tpu_sc as plsc

NUM_GRAPHS = 64
REUSE_WEIGHT = 0.01

NC = 2
NS = 16
L = 16
NW = NC * NS
ROWS = 100000
D = 128
RPW = ROWS // NW
CHUNK = 125
NCHUNK = RPW // CHUNK
NCHUNK_PAD = 2 * L
SEG_PAD = NUM_GRAPHS + 1
NBUF = 6
RUNROLL = 5


def _seg_body(codes_hbm, batch_hbm, sums_out, cnts_out,
              idx_v, bufs, ones_v, csum_v, c125_v, fid_v,
              sums_sh, cnts_sh, load_sems, idx_sem):
    c = lax.axis_index("c")
    s = lax.axis_index("s")
    wid = s * NC + c
    base = wid * RPW

    zvec = jnp.zeros((L,), jnp.float32)

    def load(j):
        pltpu.async_copy(codes_hbm.at[pl.ds(base + j * CHUNK, CHUNK)],
                         bufs.at[j % NBUF],
                         load_sems.at[j % NBUF])

    def wait_load(j):
        pltpu.make_async_copy(codes_hbm.at[pl.ds(base + j * CHUNK, CHUNK)],
                              bufs.at[j % NBUF],
                              load_sems.at[j % NBUF]).wait()

    for j in range(NBUF - 1):
        load(j)
    idx_cp = pltpu.async_copy(batch_hbm.at[pl.ds(wid * NCHUNK, NCHUNK)],
                              idx_v, idx_sem)

    for i in range(4):
        for jj in range(D // L):
            csum_v[i, pl.ds(jj * L, L)] = zvec
        c125_v[i, :] = zvec
    pltpu.sync_copy(csum_v.at[pl.ds(0, 4)], sums_sh.at[pl.ds(s * 4, 4)])

    @pl.when(s == 0)
    def _init():
        pltpu.sync_copy(csum_v.at[pl.ds(0, 1)],
                        sums_sh.at[pl.ds(NUM_GRAPHS, 1)])
        def zrow(i, carry):
            ones_v[i, :] = zvec
            return carry
        lax.fori_loop(0, SEG_PAD, zrow, 0)
        pltpu.sync_copy(ones_v.at[pl.ds(0, SEG_PAD)], cnts_sh)

    plsc.subcore_barrier()

    ovec = jnp.ones((L,), jnp.float32)

    def orow(i, carry):
        ones_v[i, :] = ovec
        return carry
    lax.fori_loop(0, CHUNK, orow, 0)

    cvec = jnp.full((L,), float(CHUNK), jnp.float32)
    for i in range(NCHUNK_PAD):
        c125_v[i, :] = cvec

    idx_cp.wait()

    lane_iota = lax.iota(jnp.int32, L)
    mark = jnp.full((L,), NUM_GRAPHS, jnp.int32)
    fids = [None, mark]

    for j in range(NCHUNK):
        if j + NBUF - 1 < NCHUNK:
            load(j + NBUF - 1)
        wait_load(j)
        buf = bufs.at[j % NBUF]

        vmin = idx_v[j, pl.ds(0, L)]
        vmax = vmin
        for off in (16, 32, 48, 64, 80, 96, 109):
            sl = idx_v[j, pl.ds(off, L)]
            vmin = jnp.minimum(vmin, sl)
            vmax = jnp.maximum(vmax, sl)
        smin = lax.reduce_min(vmin, axes=(0,))
        smax = lax.reduce_max(vmax, axes=(0,))
        mixed = smin != smax
        fid_j = jnp.where(mixed, NUM_GRAPHS, smin)
        half, lane = divmod(j, L)
        if fids[half] is None:
            fids[half] = jnp.where(lane_iota == lane, fid_j, 0)
        else:
            fids[half] = jnp.where(lane_iota == lane, fid_j, fids[half])

        @pl.when(mixed)
        def _fallback():
            pltpu.sync_copy(buf, sums_sh.at[idx_v.at[j]], add=True)
            pltpu.sync_copy(ones_v, cnts_sh.at[idx_v.at[j]], add=True)

        @pl.when(jnp.logical_not(mixed))
        def _reduce():
            def rbody(r5, accs):
                accs = list(accs)
                for rr in range(RUNROLL):
                    r = r5 * RUNROLL + rr
                    for jj in range(D // L):
                        accs[jj] = accs[jj] + buf[r, pl.ds(jj * L, L)]
                return tuple(accs)
            accs = lax.fori_loop(0, CHUNK // RUNROLL, rbody,
                                 tuple(zvec for _ in range(D // L)))
            for jj in range(D // L):
                csum_v[j, pl.ds(jj * L, L)] = accs[jj]

    fid_v[pl.ds(0, L)] = fids[0]
    fid_v[pl.ds(L, L)] = fids[1]
    pltpu.sync_copy(csum_v, sums_sh.at[fid_v], add=True)
    pltpu.sync_copy(c125_v, cnts_sh.at[fid_v], add=True)

    plsc.subcore_barrier()

    @pl.when(s == 0)
    def _flush():
        pltpu.sync_copy(sums_sh, bufs.at[0].at[pl.ds(0, SEG_PAD)])
        pltpu.sync_copy(bufs.at[0].at[pl.ds(0, SEG_PAD)], sums_out.at[c])
        pltpu.sync_copy(cnts_sh, ones_v.at[pl.ds(0, SEG_PAD)])
        pltpu.sync_copy(ones_v.at[pl.ds(0, SEG_PAD)], cnts_out.at[c])


@functools.lru_cache(maxsize=1)
def _make_seg_reduce():
    return functools.partial(
        pl.kernel,
        out_type=[
            jax.ShapeDtypeStruct((NC, SEG_PAD, D), jnp.float32),
            jax.ShapeDtypeStruct((NC, SEG_PAD, L), jnp.float32),
        ],
        mesh=plsc.VectorSubcoreMesh(core_axis_name="c", subcore_axis_name="s"),
        scratch_types=[
            pltpu.VMEM((NCHUNK, CHUNK), jnp.int32),
            pltpu.VMEM((NBUF, CHUNK, D), jnp.float32),
            pltpu.VMEM((CHUNK, L), jnp.float32),
            pltpu.VMEM((NCHUNK_PAD, D), jnp.float32),
            pltpu.VMEM((NCHUNK_PAD, L), jnp.float32),
            pltpu.VMEM((NCHUNK_PAD,), jnp.int32),
            pltpu.VMEM_SHARED((SEG_PAD, D), jnp.float32),
            pltpu.VMEM_SHARED((SEG_PAD, L), jnp.float32),
            pltpu.SemaphoreType.DMA((NBUF,)),
            pltpu.SemaphoreType.DMA,
        ],
        compiler_params=pltpu.CompilerParams(use_tc_tiling_on_sc=False,
                                             needs_layout_passes=False),
    )(_seg_body)


def _fin_body(s_ref, c_ref, o_ref):
    sums = s_ref[0, :NUM_GRAPHS, :] + s_ref[1, :NUM_GRAPHS, :]
    counts = c_ref[0, :NUM_GRAPHS, 0:1] + c_ref[1, :NUM_GRAPHS, 0:1]
    means = sums / counts
    mu = jnp.mean(means, axis=0, keepdims=True)
    dev = means - mu
    var = jnp.sum(dev * dev, axis=0) / (NUM_GRAPHS - 1)
    o_ref[...] = jnp.reshape(-REUSE_WEIGHT * jnp.mean(var), (1, 1))


def kernel(sparse_codes, batch):
    batch2d = batch.astype(jnp.int32).reshape(NW * NCHUNK, CHUNK)
    sums, cnts = _make_seg_reduce()(sparse_codes, batch2d)
    out = pl.pallas_call(
        _fin_body,
        out_shape=jax.ShapeDtypeStruct((1, 1), jnp.float32),
    )(sums, cnts)
    return out[0, 0]


# speedup vs baseline: 1.1807x; 1.1580x over previous
import functools

import jax
import jax.numpy as jnp
from jax import lax
from jax.experimental import pallas as pl
from jax.experimental.pallas import tpu as pltpu
from jax.experimental.pallas import tpu_sc as plsc

NUM_GRAPHS = 64
REUSE_WEIGHT = 0.01

NC = 2
NS = 16
L = 16
NW = NC * NS
ROWS = 100000
D = 128
RPW = ROWS // NW
CHUNK = 125
NCHUNK = RPW // CHUNK
NCHUNK_PAD = 2 * L
SEG_PAD = NUM_GRAPHS + 1
NBUF = 6
RUNROLL = 5


def _seg_body(codes_hbm, batch_hbm, sums_out, cnts_out,
              idx_v, bufs, ones_v, csum_v, c125_v, fid_v,
              sums_sh, cnts_sh, load_sems, idx_sem):
    c = lax.axis_index("c")
    s = lax.axis_index("s")
    wid = s * NC + c
    base = wid * RPW

    zvec = jnp.zeros((L,), jnp.float32)

    def load(j):
        pltpu.async_copy(codes_hbm.at[pl.ds(base + j * CHUNK, CHUNK)],
                         bufs.at[j % NBUF],
                         load_sems.at[j % NBUF])

    def wait_load(j):
        pltpu.make_async_copy(codes_hbm.at[pl.ds(base + j * CHUNK, CHUNK)],
                              bufs.at[j % NBUF],
                              load_sems.at[j % NBUF]).wait()

    for j in range(NBUF - 1):
        load(j)
    idx_cp = pltpu.async_copy(batch_hbm.at[pl.ds(wid * NCHUNK, NCHUNK)],
                              idx_v, idx_sem)

    for i in range(4):
        for jj in range(D // L):
            csum_v[i, pl.ds(jj * L, L)] = zvec
        c125_v[i, :] = zvec
    pltpu.sync_copy(csum_v.at[pl.ds(0, 4)], sums_sh.at[pl.ds(s * 4, 4)])

    @pl.when(s == 0)
    def _init():
        pltpu.sync_copy(csum_v.at[pl.ds(0, 1)],
                        sums_sh.at[pl.ds(NUM_GRAPHS, 1)])
        def zrow(i, carry):
            ones_v[i, :] = zvec
            return carry
        lax.fori_loop(0, SEG_PAD, zrow, 0)
        pltpu.sync_copy(ones_v.at[pl.ds(0, SEG_PAD)], cnts_sh)

    plsc.subcore_barrier()

    ovec = jnp.ones((L,), jnp.float32)

    def orow(i, carry):
        ones_v[i, :] = ovec
        return carry
    lax.fori_loop(0, CHUNK, orow, 0)

    cvec = jnp.full((L,), float(CHUNK), jnp.float32)
    for i in range(NCHUNK_PAD):
        c125_v[i, :] = cvec

    idx_cp.wait()

    lane_iota = lax.iota(jnp.int32, L)
    mark = jnp.full((L,), NUM_GRAPHS, jnp.int32)

    def chunk_body(j, carry):
        f0, f1 = carry
        if True:
            @pl.when(j + NBUF - 1 < NCHUNK)
            def _prefetch():
                load(j + NBUF - 1)
        wait_load(j)
        buf = bufs.at[j % NBUF]

        vmin = idx_v[j, pl.ds(0, L)]
        vmax = vmin
        for off in (16, 32, 48, 64, 80, 96, 109):
            sl = idx_v[j, pl.ds(off, L)]
            vmin = jnp.minimum(vmin, sl)
            vmax = jnp.maximum(vmax, sl)
        smin = lax.reduce_min(vmin, axes=(0,))
        smax = lax.reduce_max(vmax, axes=(0,))
        mixed = smin != smax
        fid_j = jnp.where(mixed, NUM_GRAPHS, smin)
        lane = j % L
        upd = lane_iota == lane
        f0 = jnp.where(jnp.logical_and(j < L, upd), fid_j, f0)
        f1 = jnp.where(jnp.logical_and(j >= L, upd), fid_j, f1)

        @pl.when(mixed)
        def _fallback():
            pltpu.sync_copy(buf, sums_sh.at[idx_v.at[j]], add=True)
            pltpu.sync_copy(ones_v, cnts_sh.at[idx_v.at[j]], add=True)

        @pl.when(jnp.logical_not(mixed))
        def _reduce():
            def rbody(r5, accs):
                accs = list(accs)
                for rr in range(RUNROLL):
                    r = r5 * RUNROLL + rr
                    for jj in range(D // L):
                        accs[jj] = accs[jj] + buf[r, pl.ds(jj * L, L)]
                return tuple(accs)
            accs = lax.fori_loop(0, CHUNK // RUNROLL, rbody,
                                 tuple(zvec for _ in range(D // L)))
            for jj in range(D // L):
                csum_v[j, pl.ds(jj * L, L)] = accs[jj]
        return f0, f1

    f0, f1 = lax.fori_loop(0, NCHUNK, chunk_body,
                           (jnp.zeros((L,), jnp.int32), mark))
    fid_v[pl.ds(0, L)] = f0
    fid_v[pl.ds(L, L)] = f1
    pltpu.sync_copy(csum_v, sums_sh.at[fid_v], add=True)
    pltpu.sync_copy(c125_v, cnts_sh.at[fid_v], add=True)

    plsc.subcore_barrier()

    @pl.when(s == 0)
    def _flush():
        pltpu.sync_copy(sums_sh, bufs.at[0].at[pl.ds(0, SEG_PAD)])
        pltpu.sync_copy(bufs.at[0].at[pl.ds(0, SEG_PAD)], sums_out.at[c])
        pltpu.sync_copy(cnts_sh, ones_v.at[pl.ds(0, SEG_PAD)])
        pltpu.sync_copy(ones_v.at[pl.ds(0, SEG_PAD)], cnts_out.at[c])


@functools.lru_cache(maxsize=1)
def _make_seg_reduce():
    return functools.partial(
        pl.kernel,
        out_type=[
            jax.ShapeDtypeStruct((NC, SEG_PAD, D), jnp.float32),
            jax.ShapeDtypeStruct((NC, SEG_PAD, L), jnp.float32),
        ],
        mesh=plsc.VectorSubcoreMesh(core_axis_name="c", subcore_axis_name="s"),
        scratch_types=[
            pltpu.VMEM((NCHUNK, CHUNK), jnp.int32),
            pltpu.VMEM((NBUF, CHUNK, D), jnp.float32),
            pltpu.VMEM((CHUNK, L), jnp.float32),
            pltpu.VMEM((NCHUNK_PAD, D), jnp.float32),
            pltpu.VMEM((NCHUNK_PAD, L), jnp.float32),
            pltpu.VMEM((NCHUNK_PAD,), jnp.int32),
            pltpu.VMEM_SHARED((SEG_PAD, D), jnp.float32),
            pltpu.VMEM_SHARED((SEG_PAD, L), jnp.float32),
            pltpu.SemaphoreType.DMA((NBUF,)),
            pltpu.SemaphoreType.DMA,
        ],
        compiler_params=pltpu.CompilerParams(use_tc_tiling_on_sc=False,
                                             needs_layout_passes=False),
    )(_seg_body)


def _fin_body(s_hbm, c_hbm, o_ref, s_v, c_v, sem_s, sem_c):
    cp_s = pltpu.make_async_copy(s_hbm, s_v, sem_s)
    cp_c = pltpu.make_async_copy(c_hbm, c_v, sem_c)
    cp_s.start()
    cp_c.start()
    cp_s.wait()
    cp_c.wait()
    sums = s_v[0, :NUM_GRAPHS, :] + s_v[1, :NUM_GRAPHS, :]
    counts = c_v[0, :NUM_GRAPHS, 0:1] + c_v[1, :NUM_GRAPHS, 0:1]
    means = sums / counts
    mu = jnp.mean(means, axis=0, keepdims=True)
    dev = means - mu
    var = jnp.sum(dev * dev, axis=0) / (NUM_GRAPHS - 1)
    o_ref[...] = jnp.reshape(-REUSE_WEIGHT * jnp.mean(var), (1, 1))


def kernel(sparse_codes, batch):
    batch2d = batch.astype(jnp.int32).reshape(NW * NCHUNK, CHUNK)
    sums, cnts = _make_seg_reduce()(sparse_codes, batch2d)
    out = pl.pallas_call(
        _fin_body,
        in_specs=[pl.BlockSpec(memory_space=pltpu.MemorySpace.HBM),
                  pl.BlockSpec(memory_space=pltpu.MemorySpace.HBM)],
        out_shape=jax.ShapeDtypeStruct((1, 1), jnp.float32),
        scratch_shapes=[
            pltpu.VMEM((NC, SEG_PAD, D), jnp.float32),
            pltpu.VMEM((NC, SEG_PAD, L), jnp.float32),
            pltpu.SemaphoreType.DMA,
            pltpu.SemaphoreType.DMA,
        ],
    )(sums, cnts)
    return out[0, 0]
